# Initial kernel scaffold; baseline (speedup 1.0000x reference)
import jax
import jax.numpy as jnp
from jax.experimental import pallas as pl


def kernel(x, weight1, weight2):
    raise NotImplementedError("write your pallas kernel here")


# speedup vs baseline: 7.7463x; 7.7463x over previous
import functools

import jax
import jax.numpy as jnp
from jax import lax
from jax.experimental import pallas as pl
from jax.experimental.pallas import tpu as pltpu
from jax.experimental.pallas import tpu_sc as plsc


def _make_sc_gather(V, F, N):
    info = plsc.get_sparse_core_info()
    NC, NS = info.num_cores, info.num_subcores
    NW = NC * NS
    assert N % NW == 0
    BPW = N // NW
    R = 104
    assert BPW % R == 0
    CH = BPW // R
    NBUF = 4
    assert CH % NBUF == 0
    T = CH // NBUF

    mesh = plsc.VectorSubcoreMesh(core_axis_name="c", subcore_axis_name="s")

    @functools.partial(
        pl.kernel,
        mesh=mesh,
        out_type=jax.ShapeDtypeStruct((N, F), jnp.float32),
        scratch_types=[
            pltpu.VMEM((BPW,), jnp.int32),
            pltpu.VMEM((NBUF, R, F), jnp.float32),
            pltpu.SemaphoreType.DMA((NBUF,)),
            pltpu.SemaphoreType.DMA((NBUF,)),
        ],
    )
    def gather_kernel(table, idx, out, idx_v, rows_v, gsem, osem):
        wid = lax.axis_index("s") * NC + lax.axis_index("c")
        base = wid * BPW
        pltpu.sync_copy(idx.at[pl.ds(base, BPW)], idx_v)

        def g_copy(c, b):
            return pltpu.make_async_copy(
                table.at[idx_v.at[pl.ds(c * R, R)]], rows_v.at[b], gsem.at[b])

        def s_copy(c, b):
            return pltpu.make_async_copy(
                rows_v.at[b], out.at[pl.ds(base + c * R, R)], osem.at[b])

        for b in range(NBUF):
            g_copy(b, b).start()

        def body(t, carry):
            for b in range(NBUF):
                c = t * NBUF + b
                g_copy(c, b).wait()
                s_copy(c, b).start()
                nc = c + NBUF

                @pl.when(nc < CH)
                def _():
                    s_copy(c, b).wait()
                    g_copy(nc, b).start()
            return carry

        lax.fori_loop(0, T, body, 0)
        for b in range(NBUF):
            s_copy(CH - NBUF + b, b).wait()

    return gather_kernel


def _make_tc_project(N, F, E, BLK=1024):
    assert N % BLK == 0

    def mm_kernel(emb_ref, w2_ref, out_ref):
        out_ref[...] = lax.dot_general(
            emb_ref[...], w2_ref[...],
            dimension_numbers=(((1,), (1,)), ((), ())),
            preferred_element_type=jnp.float32)

    return pl.pallas_call(
        mm_kernel,
        grid=(N // BLK,),
        in_specs=[
            pl.BlockSpec((BLK, F), lambda i: (i, 0)),
            pl.BlockSpec((E, F), lambda i: (0, 0)),
        ],
        out_specs=pl.BlockSpec((BLK, E), lambda i: (i, 0)),
        out_shape=jax.ShapeDtypeStruct((N, E), jnp.float32),
    )


def kernel(x, weight1, weight2):
    B1, B2 = x.shape
    V, F = weight1.shape
    E = weight2.shape[0]
    N = B1 * B2
    idx = x.reshape(N).astype(jnp.int32)
    emb = _make_sc_gather(V, F, N)(weight1, idx)
    out = _make_tc_project(N, F, E)(emb, weight2)
    return out.reshape(B1, B2, E)


# speedup vs baseline: 12.2907x; 1.5866x over previous
import functools

import jax
import jax.numpy as jnp
from jax import lax
from jax.experimental import pallas as pl
from jax.experimental.pallas import tpu as pltpu
from jax.experimental.pallas import tpu_sc as plsc


def _make_sc_gather(V, F, N):
    info = plsc.get_sparse_core_info()
    NC, NS = info.num_cores, info.num_subcores
    NW = NC * NS
    assert N % NW == 0
    BPW = N // NW
    R = 104
    assert BPW % R == 0
    CH = BPW // R
    NBUF = 4
    assert CH % NBUF == 0
    T = CH // NBUF

    mesh = plsc.VectorSubcoreMesh(core_axis_name="c", subcore_axis_name="s")

    @functools.partial(
        pl.kernel,
        mesh=mesh,
        out_type=jax.ShapeDtypeStruct((N, F), jnp.float32),
        scratch_types=[
            pltpu.VMEM((BPW,), jnp.int32),
            pltpu.VMEM((NBUF, R, F), jnp.float32),
            pltpu.SemaphoreType.DMA((NBUF,)),
            pltpu.SemaphoreType.DMA((NBUF,)),
        ],
    )
    def gather_kernel(table, idx, out, idx_v, rows_v, gsem, osem):
        wid = lax.axis_index("s") * NC + lax.axis_index("c")
        base = wid * BPW
        pltpu.sync_copy(idx.at[pl.ds(base, BPW)], idx_v)

        def g_copy(c, b):
            return pltpu.make_async_copy(
                table.at[idx_v.at[pl.ds(c * R, R)]], rows_v.at[b], gsem.at[b])

        def s_copy(c, b):
            return pltpu.make_async_copy(
                rows_v.at[b], out.at[pl.ds(base + c * R, R)], osem.at[b])

        for b in range(NBUF):
            g_copy(b, b).start()

        def body(t, carry):
            for b in range(NBUF):
                c = t * NBUF + b
                g_copy(c, b).wait()
                s_copy(c, b).start()
                nc = c + NBUF

                @pl.when(nc < CH)
                def _():
                    s_copy(c, b).wait()
                    g_copy(nc, b).start()
            return carry

        lax.fori_loop(0, T, body, 0)
        for b in range(NBUF):
            s_copy(CH - NBUF + b, b).wait()

    return gather_kernel


def _make_tc_project(B1, B2, F, E, BA=128):
    assert B1 % BA == 0
    N = B1 * B2

    def mm_kernel(emb_ref, w2_ref, out_ref):
        r = lax.dot_general(
            emb_ref[...], w2_ref[...],
            dimension_numbers=(((1,), (1,)), ((), ())),
            preferred_element_type=jnp.float32)
        out_ref[...] = r.reshape(BA, B2, E)

    return pl.pallas_call(
        mm_kernel,
        grid=(B1 // BA,),
        in_specs=[
            pl.BlockSpec((BA * B2, F), lambda i: (i, 0)),
            pl.BlockSpec((E, F), lambda i: (0, 0)),
        ],
        out_specs=pl.BlockSpec((BA, B2, E), lambda i: (i, 0, 0)),
        out_shape=jax.ShapeDtypeStruct((B1, B2, E), jnp.float32),
    )


def kernel(x, weight1, weight2):
    B1, B2 = x.shape
    V, F = weight1.shape
    E = weight2.shape[0]
    N = B1 * B2
    idx = x.reshape(N).astype(jnp.int32)
    emb = _make_sc_gather(V, F, N)(weight1, idx)
    return _make_tc_project(B1, B2, F, E)(emb, weight2)


# speedup vs baseline: 12.6660x; 1.0305x over previous
import functools

import jax
import jax.numpy as jnp
from jax import lax
from jax.experimental import pallas as pl
from jax.experimental.pallas import tpu as pltpu
from jax.experimental.pallas import tpu_sc as plsc


def _make_sc_gather(V, F, N):
    info = plsc.get_sparse_core_info()
    NC, NS = info.num_cores, info.num_subcores
    NW = NC * NS
    assert N % NW == 0
    BPW = N // NW
    R = 104
    assert BPW % R == 0
    CH = BPW // R
    NBUF = 4
    assert CH % NBUF == 0
    T = CH // NBUF

    mesh = plsc.VectorSubcoreMesh(core_axis_name="c", subcore_axis_name="s")

    @functools.partial(
        pl.kernel,
        mesh=mesh,
        out_type=jax.ShapeDtypeStruct((N, F), jnp.float32),
        scratch_types=[
            pltpu.VMEM((BPW,), jnp.int32),
            pltpu.VMEM((NBUF, R, F), jnp.float32),
            pltpu.SemaphoreType.DMA((NBUF,)),
            pltpu.SemaphoreType.DMA((NBUF,)),
        ],
    )
    def gather_kernel(table, idx, out, idx_v, rows_v, gsem, osem):
        wid = lax.axis_index("s") * NC + lax.axis_index("c")
        base = wid * BPW
        pltpu.sync_copy(idx.at[pl.ds(base, BPW)], idx_v)

        def g_copy(c, b):
            return pltpu.make_async_copy(
                table.at[idx_v.at[pl.ds(c * R, R)]], rows_v.at[b], gsem.at[b])

        def s_copy(c, b):
            return pltpu.make_async_copy(
                rows_v.at[b], out.at[pl.ds(base + c * R, R)], osem.at[b])

        for b in range(NBUF):
            g_copy(b, b).start()

        def body(t, carry):
            for b in range(NBUF):
                c = t * NBUF + b
                g_copy(c, b).wait()
                s_copy(c, b).start()
                nc = c + NBUF

                @pl.when(nc < CH)
                def _():
                    s_copy(c, b).wait()
                    g_copy(nc, b).start()
            return carry

        lax.fori_loop(0, T, body, 0)
        for b in range(NBUF):
            s_copy(CH - NBUF + b, b).wait()

    return gather_kernel


def _make_tc_project_chunk(B1, B2, F, E, B1K, k0, BA=128):
    assert B1K % BA == 0

    def mm_kernel(emb_ref, w2_ref, *rest):
        out_ref = rest[-1]
        r = lax.dot_general(
            emb_ref[...], w2_ref[...],
            dimension_numbers=(((1,), (1,)), ((), ())),
            preferred_element_type=jnp.float32)
        out_ref[...] = r.reshape(BA, B2, E)

    in_specs = [
        pl.BlockSpec((BA * B2, F), lambda i: (i, 0)),
        pl.BlockSpec((E, F), lambda i: (0, 0)),
    ]
    aliased = k0 > 0
    if aliased:
        in_specs.append(pl.BlockSpec(memory_space=pl.ANY))

    return pl.pallas_call(
        mm_kernel,
        grid=(B1K // BA,),
        in_specs=in_specs,
        out_specs=pl.BlockSpec((BA, B2, E), lambda i: (k0 // BA + i, 0, 0)),
        out_shape=jax.ShapeDtypeStruct((B1, B2, E), jnp.float32),
        input_output_aliases={2: 0} if aliased else {},
    )


def kernel(x, weight1, weight2):
    B1, B2 = x.shape
    V, F = weight1.shape
    E = weight2.shape[0]
    N = B1 * B2
    idx = x.reshape(N).astype(jnp.int32)

    K = 4
    B1K = B1 // K
    NK = N // K
    gather = _make_sc_gather(V, F, NK)
    embs = [gather(weight1, lax.slice(idx, [k * NK], [(k + 1) * NK]))
            for k in range(K)]
    out = None
    for k in range(K):
        mm = _make_tc_project_chunk(B1, B2, F, E, B1K, k * B1K)
        args = (embs[k], weight2) if out is None else (embs[k], weight2, out)
        out = mm(*args)
    return out
